# baseline (speedup 1.0000x reference)
import jax
import jax.numpy as jnp
from jax import lax
from jax.experimental import pallas as pl
from jax.experimental.pallas import tpu as pltpu
from jax.experimental.pallas import tpu_sc as plsc

VOCAB = 1000000
D = 64
BATCH = 4096
HIST = 200
B = BATCH * HIST

NC, NS = 2, 16
NW = NC * NS

RBLK = 128
RNBLK = VOCAB // RBLK
RPART = VOCAB - RNBLK * RBLK
RKFULL = RNBLK // NW
RLEFT = RNBLK - RKFULL * NW


def _repack_body(tabT_hbm, tailT_hbm, wide_hbm, buf_v, wbuf_v, tail_v,
                 sem_i, sem_o):
    wid = lax.axis_index("s") * NC + lax.axis_index("c")
    lane = lax.iota(jnp.int32, 16)
    dvecs = [lane + g * 16 for g in range(4)]

    def col(k):
        return (wid + NW * k) * RBLK

    def load(c0, slot):
        pltpu.async_copy(tabT_hbm.at[:, pl.ds(c0, RBLK)],
                         buf_v.at[slot], sem_i.at[slot])

    def wait_load(slot):
        pltpu.make_async_copy(tabT_hbm.at[:, pl.ds(0, RBLK)],
                              buf_v.at[slot], sem_i.at[slot]).wait()

    def transpose(slot):
        def jbody(j, carry):
            jvec = jnp.full((16,), j, jnp.int32)
            for g in range(4):
                v = plsc.load_gather(buf_v.at[slot], [dvecs[g], jvec])
                wbuf_v[slot, j, pl.ds(g * 16, 16)] = v
            return carry

        lax.fori_loop(0, RBLK, jbody, 0)

    def store(c0, slot):
        pltpu.async_copy(wbuf_v.at[slot], wide_hbm.at[pl.ds(c0, RBLK)],
                         sem_o.at[slot])

    def wait_store(slot):
        pltpu.make_async_copy(wbuf_v.at[slot], wide_hbm.at[pl.ds(0, RBLK)],
                              sem_o.at[slot]).wait()

    load(col(0), 0)
    load(col(1), 1)

    def body(p, carry):
        k0 = p * 2
        wait_load(0)

        @pl.when(p > 0)
        def _w0():
            wait_store(0)

        transpose(0)
        store(col(k0), 0)

        @pl.when(p < RKFULL // 2 - 1)
        def _f0():
            load(col(k0 + 2), 0)

        wait_load(1)

        @pl.when(p > 0)
        def _w1():
            wait_store(1)

        transpose(1)
        store(col(k0 + 1), 1)

        @pl.when(p < RKFULL // 2 - 1)
        def _f1():
            load(col(k0 + 3), 1)

        return carry

    lax.fori_loop(0, RKFULL // 2, body, 0)
    wait_store(0)
    wait_store(1)

    @pl.when(wid < RLEFT)
    def _tail_full():
        c0 = (RKFULL * NW + wid) * RBLK
        load(c0, 0)
        wait_load(0)
        transpose(0)
        store(c0, 0)
        wait_store(0)

    @pl.when(wid == RLEFT)
    def _tail_part():
        pltpu.sync_copy(tailT_hbm, tail_v)

        def jbody(j, carry):
            jvec = jnp.full((16,), j, jnp.int32)
            for g in range(4):
                v = plsc.load_gather(tail_v, [dvecs[g], jvec])
                wbuf_v[0, j, pl.ds(g * 16, 16)] = v
            return carry

        lax.fori_loop(0, RPART, jbody, 0)
        pltpu.async_copy(wbuf_v.at[0, pl.ds(0, RPART)],
                         wide_hbm.at[pl.ds(RNBLK * RBLK, RPART)],
                         sem_o.at[0])
        pltpu.make_async_copy(wbuf_v.at[0, pl.ds(0, RPART)],
                              wide_hbm.at[pl.ds(0, RPART)],
                              sem_o.at[0]).wait()


PER_W = B // NW
CHUNK = 200
N_CHUNKS = PER_W // CHUNK
NPAIR = N_CHUNKS // 2


def _gather_body(xf_hbm, wide_hbm, out_hbm, idx_v, rows_v, out_v, sem_g, sem_o):
    wid = lax.axis_index("s") * NC + lax.axis_index("c")
    base = wid * PER_W
    pltpu.sync_copy(xf_hbm.at[pl.ds(base, PER_W)], idx_v)

    def gather(c, slot):
        idx_slice = idx_v.at[pl.ds(c * CHUNK, CHUNK)]
        pltpu.async_copy(wide_hbm.at[idx_slice], rows_v.at[slot],
                         sem_g.at[slot])

    def wait_gather(slot):
        pltpu.make_async_copy(wide_hbm.at[idx_v.at[pl.ds(0, CHUNK)]],
                              rows_v.at[slot], sem_g.at[slot]).wait()

    def compact(slot):
        def rbody(r, carry):
            for k in range(4):
                out_v[slot, r, pl.ds(k * 16, 16)] = (
                    rows_v[slot, r, pl.ds(k * 16, 16)])
            return carry

        lax.fori_loop(0, CHUNK, rbody, 0)

    def store(c, slot):
        pltpu.async_copy(out_v.at[slot],
                         out_hbm.at[pl.ds(base + c * CHUNK, CHUNK)],
                         sem_o.at[slot])

    def wait_store(slot):
        pltpu.make_async_copy(out_v.at[slot],
                              out_hbm.at[pl.ds(0, CHUNK)],
                              sem_o.at[slot]).wait()

    gather(0, 0)
    gather(1, 1)

    def body(p, carry):
        c0 = p * 2
        wait_gather(0)

        @pl.when(p > 0)
        def _w0():
            wait_store(0)

        compact(0)
        store(c0, 0)

        @pl.when(p < NPAIR - 1)
        def _f0():
            gather(c0 + 2, 0)

        wait_gather(1)

        @pl.when(p > 0)
        def _w1():
            wait_store(1)

        compact(1)
        store(c0 + 1, 1)

        @pl.when(p < NPAIR - 1)
        def _f1():
            gather(c0 + 3, 1)

        return carry

    lax.fori_loop(0, NPAIR, body, 0)
    wait_store(0)
    wait_store(1)


_PARAMS = pltpu.CompilerParams(use_tc_tiling_on_sc=True,
                               needs_layout_passes=False)


@jax.jit
def _embed(xf, tabT, tailT):
    mesh = plsc.VectorSubcoreMesh(core_axis_name="c", subcore_axis_name="s")
    repack = pl.kernel(
        _repack_body,
        out_type=jax.ShapeDtypeStruct((VOCAB, 2 * D), jnp.float32),
        mesh=mesh,
        scratch_types=[
            pltpu.VMEM((2, D, RBLK), jnp.float32),
            pltpu.VMEM((2, RBLK, 2 * D), jnp.float32),
            pltpu.VMEM((D, D), jnp.float32),
            pltpu.SemaphoreType.DMA((2,)),
            pltpu.SemaphoreType.DMA((2,)),
        ],
        compiler_params=_PARAMS,
    )
    wide = repack(tabT, tailT)
    gather = pl.kernel(
        _gather_body,
        out_type=jax.ShapeDtypeStruct((B, D), jnp.float32),
        mesh=mesh,
        scratch_types=[
            pltpu.VMEM((PER_W,), jnp.int32),
            pltpu.VMEM((2, CHUNK, 2 * D), jnp.float32),
            pltpu.VMEM((2, CHUNK, D), jnp.float32),
            pltpu.SemaphoreType.DMA((2,)),
            pltpu.SemaphoreType.DMA((2,)),
        ],
        compiler_params=_PARAMS,
    )
    return gather(xf, wide)


def kernel(x, table):
    xf = x.reshape(-1).astype(jnp.int32)
    tabT = jnp.transpose(table)
    tailT = jnp.transpose(table[RNBLK * RBLK:])
    out = _embed(xf, tabT, tailT)
    return out.reshape(BATCH, HIST, D)


# speedup vs baseline: 1.9476x; 1.9476x over previous
import jax
import jax.numpy as jnp
from jax import lax
from jax.experimental import pallas as pl
from jax.experimental.pallas import tpu as pltpu
from jax.experimental.pallas import tpu_sc as plsc

VOCAB = 1000000
D = 64
BATCH = 4096
HIST = 200
B = BATCH * HIST

NC, NS = 2, 16
NW = NC * NS

RCHUNK = 256
RFULL = VOCAB // RCHUNK
RREM = VOCAB - RFULL * RCHUNK
RK = RFULL // NW


def _repack_body(tab_hbm, wide_hbm, buf_v, wbuf_v, sem_i, sem_o):
    wid = lax.axis_index("s") * NC + lax.axis_index("c")

    def off(k):
        return (wid + NW * k) * RCHUNK

    def load(r, n, slot):
        pltpu.async_copy(tab_hbm.at[pl.ds(r, n)],
                         buf_v.at[slot, pl.ds(0, n)], sem_i.at[slot])

    def wait_load(n, slot):
        pltpu.make_async_copy(tab_hbm.at[pl.ds(0, n)],
                              buf_v.at[slot, pl.ds(0, n)],
                              sem_i.at[slot]).wait()

    def widen(n, slot):
        def rbody(r, carry):
            for k in range(4):
                wbuf_v[slot, r, pl.ds(k * 16, 16)] = (
                    buf_v[slot, r, pl.ds(k * 16, 16)])
            return carry

        lax.fori_loop(0, n, rbody, 0)

    def store(r, n, slot):
        pltpu.async_copy(wbuf_v.at[slot, pl.ds(0, n)],
                         wide_hbm.at[pl.ds(r, n)],
                         sem_o.at[slot])

    def wait_store(n, slot):
        pltpu.make_async_copy(wbuf_v.at[slot, pl.ds(0, n)],
                              wide_hbm.at[pl.ds(0, n)],
                              sem_o.at[slot]).wait()

    load(off(0), RCHUNK, 0)
    load(off(1), RCHUNK, 1)

    def body(p, carry):
        k0 = p * 2
        wait_load(RCHUNK, 0)

        @pl.when(p > 0)
        def _w0():
            wait_store(RCHUNK, 0)

        widen(RCHUNK, 0)
        store(off(k0), RCHUNK, 0)

        @pl.when(p < RK // 2 - 1)
        def _f0():
            load(off(k0 + 2), RCHUNK, 0)

        wait_load(RCHUNK, 1)

        @pl.when(p > 0)
        def _w1():
            wait_store(RCHUNK, 1)

        widen(RCHUNK, 1)
        store(off(k0 + 1), RCHUNK, 1)

        @pl.when(p < RK // 2 - 1)
        def _f1():
            load(off(k0 + 3), RCHUNK, 1)

        return carry

    lax.fori_loop(0, RK // 2, body, 0)
    wait_store(RCHUNK, 0)
    wait_store(RCHUNK, 1)

    @pl.when(wid == 0)
    def _tail_a():
        load((RFULL - 2) * RCHUNK, RCHUNK, 0)
        wait_load(RCHUNK, 0)
        widen(RCHUNK, 0)
        store((RFULL - 2) * RCHUNK, RCHUNK, 0)
        wait_store(RCHUNK, 0)

    @pl.when(wid == 1)
    def _tail_b():
        load((RFULL - 1) * RCHUNK, RCHUNK, 0)
        wait_load(RCHUNK, 0)
        widen(RCHUNK, 0)
        store((RFULL - 1) * RCHUNK, RCHUNK, 0)
        wait_store(RCHUNK, 0)

    @pl.when(wid == 2)
    def _tail_rem():
        load(RFULL * RCHUNK, RREM, 0)
        wait_load(RREM, 0)
        widen(RREM, 0)
        store(RFULL * RCHUNK, RREM, 0)
        wait_store(RREM, 0)


PER_W = B // NW
CHUNK = 200
N_CHUNKS = PER_W // CHUNK
NPAIR = N_CHUNKS // 2


def _gather_body(xf_hbm, wide_hbm, out_hbm, idx_v, rows_v, out_v, sem_g, sem_o):
    wid = lax.axis_index("s") * NC + lax.axis_index("c")
    base = wid * PER_W
    pltpu.sync_copy(xf_hbm.at[pl.ds(base, PER_W)], idx_v)

    def gather(c, slot):
        idx_slice = idx_v.at[pl.ds(c * CHUNK, CHUNK)]
        pltpu.async_copy(wide_hbm.at[idx_slice], rows_v.at[slot],
                         sem_g.at[slot])

    def wait_gather(slot):
        pltpu.make_async_copy(wide_hbm.at[idx_v.at[pl.ds(0, CHUNK)]],
                              rows_v.at[slot], sem_g.at[slot]).wait()

    def compact(slot):
        def rbody(r, carry):
            for k in range(4):
                out_v[slot, r, pl.ds(k * 16, 16)] = (
                    rows_v[slot, r, pl.ds(k * 16, 16)])
            return carry

        lax.fori_loop(0, CHUNK, rbody, 0)

    def store(c, slot):
        pltpu.async_copy(out_v.at[slot],
                         out_hbm.at[pl.ds(base + c * CHUNK, CHUNK)],
                         sem_o.at[slot])

    def wait_store(slot):
        pltpu.make_async_copy(out_v.at[slot],
                              out_hbm.at[pl.ds(0, CHUNK)],
                              sem_o.at[slot]).wait()

    gather(0, 0)
    gather(1, 1)

    def body(p, carry):
        c0 = p * 2
        wait_gather(0)

        @pl.when(p > 0)
        def _w0():
            wait_store(0)

        compact(0)
        store(c0, 0)

        @pl.when(p < NPAIR - 1)
        def _f0():
            gather(c0 + 2, 0)

        wait_gather(1)

        @pl.when(p > 0)
        def _w1():
            wait_store(1)

        compact(1)
        store(c0 + 1, 1)

        @pl.when(p < NPAIR - 1)
        def _f1():
            gather(c0 + 3, 1)

        return carry

    lax.fori_loop(0, NPAIR, body, 0)
    wait_store(0)
    wait_store(1)


_PARAMS = pltpu.CompilerParams(use_tc_tiling_on_sc=True,
                               needs_layout_passes=False)


@jax.jit
def _embed(xf, table):
    mesh = plsc.VectorSubcoreMesh(core_axis_name="c", subcore_axis_name="s")
    repack = pl.kernel(
        _repack_body,
        out_type=jax.ShapeDtypeStruct((VOCAB, 2 * D), jnp.float32),
        mesh=mesh,
        scratch_types=[
            pltpu.VMEM((2, RCHUNK, D), jnp.float32),
            pltpu.VMEM((2, RCHUNK, 2 * D), jnp.float32),
            pltpu.SemaphoreType.DMA((2,)),
            pltpu.SemaphoreType.DMA((2,)),
        ],
        compiler_params=_PARAMS,
    )
    wide = jnp.pad(table, ((0, 0), (0, D)))
    gather = pl.kernel(
        _gather_body,
        out_type=jax.ShapeDtypeStruct((B, D), jnp.float32),
        mesh=mesh,
        scratch_types=[
            pltpu.VMEM((PER_W,), jnp.int32),
            pltpu.VMEM((2, CHUNK, 2 * D), jnp.float32),
            pltpu.VMEM((2, CHUNK, D), jnp.float32),
            pltpu.SemaphoreType.DMA((2,)),
            pltpu.SemaphoreType.DMA((2,)),
        ],
        compiler_params=_PARAMS,
    )
    return gather(xf, wide)


def kernel(x, table):
    xf = x.reshape(-1).astype(jnp.int32)
    out = _embed(xf, table)
    return out.reshape(BATCH, HIST, D)


# speedup vs baseline: 1.9554x; 1.0040x over previous
import jax
import jax.numpy as jnp
from jax import lax
from jax.experimental import pallas as pl
from jax.experimental.pallas import tpu as pltpu
from jax.experimental.pallas import tpu_sc as plsc

VOCAB = 1000000
D = 64
BATCH = 4096
HIST = 200
B = BATCH * HIST

NC, NS = 2, 16
NW = NC * NS

PER_W = B // NW
CHUNK = 200
N_CHUNKS = PER_W // CHUNK
NPAIR = N_CHUNKS // 2


def _gather_body(xf_hbm, wide_hbm, out_hbm, idx_v, rows_v, out_v, sem_g, sem_o):
    wid = lax.axis_index("s") * NC + lax.axis_index("c")
    base = wid * PER_W
    pltpu.sync_copy(xf_hbm.at[pl.ds(base, PER_W)], idx_v)

    def gather(c, slot):
        idx_slice = idx_v.at[pl.ds(c * CHUNK, CHUNK)]
        pltpu.async_copy(wide_hbm.at[idx_slice], rows_v.at[slot],
                         sem_g.at[slot])

    def wait_gather(slot):
        pltpu.make_async_copy(wide_hbm.at[idx_v.at[pl.ds(0, CHUNK)]],
                              rows_v.at[slot], sem_g.at[slot]).wait()

    def compact(slot):
        def rbody(r, carry):
            for k in range(4):
                out_v[slot, r, pl.ds(k * 16, 16)] = (
                    rows_v[slot, r, pl.ds(k * 16, 16)])
            return carry

        lax.fori_loop(0, CHUNK, rbody, 0)

    def store(c, slot):
        pltpu.async_copy(out_v.at[slot],
                         out_hbm.at[pl.ds(base + c * CHUNK, CHUNK)],
                         sem_o.at[slot])

    def wait_store(slot):
        pltpu.make_async_copy(out_v.at[slot],
                              out_hbm.at[pl.ds(0, CHUNK)],
                              sem_o.at[slot]).wait()

    gather(0, 0)
    gather(1, 1)

    def body(p, carry):
        c0 = p * 2
        wait_gather(0)

        @pl.when(p > 0)
        def _w0():
            wait_store(0)

        compact(0)
        store(c0, 0)

        @pl.when(p < NPAIR - 1)
        def _f0():
            gather(c0 + 2, 0)

        wait_gather(1)

        @pl.when(p > 0)
        def _w1():
            wait_store(1)

        compact(1)
        store(c0 + 1, 1)

        @pl.when(p < NPAIR - 1)
        def _f1():
            gather(c0 + 3, 1)

        return carry

    lax.fori_loop(0, NPAIR, body, 0)
    wait_store(0)
    wait_store(1)


_PARAMS = pltpu.CompilerParams(use_tc_tiling_on_sc=True,
                               needs_layout_passes=False)


@jax.jit
def _embed(xf, table):
    mesh = plsc.VectorSubcoreMesh(core_axis_name="c", subcore_axis_name="s")
    wide = jnp.pad(table, ((0, 0), (0, D)))
    gather = pl.kernel(
        _gather_body,
        out_type=jax.ShapeDtypeStruct((B, D), jnp.float32),
        mesh=mesh,
        scratch_types=[
            pltpu.VMEM((PER_W,), jnp.int32),
            pltpu.VMEM((2, CHUNK, 2 * D), jnp.float32),
            pltpu.VMEM((2, CHUNK, D), jnp.float32),
            pltpu.SemaphoreType.DMA((2,)),
            pltpu.SemaphoreType.DMA((2,)),
        ],
        compiler_params=_PARAMS,
    )
    return gather(xf, wide)


def kernel(x, table):
    xf = x.reshape(-1).astype(jnp.int32)
    out = _embed(xf, table)
    return out.reshape(BATCH, HIST, D)
